# baseline (speedup 1.0000x reference)
import functools

import jax
import jax.numpy as jnp
import numpy as np
from jax import lax
from jax.experimental import pallas as pl
from jax.experimental.pallas import tpu as pltpu
from jax.experimental.pallas import tpu_sc as plsc

_VOCAB = 1_000_000
_VOCAB_PAD = 1_048_576
_TROWS = 8192
_TGRID = 8
_LANES = 128

_B, _N = 16384, 200
_TOTAL = _B * _N
_NC, _NS = 2, 16
_NW = _NC * _NS
_WCOLS = _B // _NW
_CROWS = 40
_CCOLS = 256
_NCHUNKS = (_N // _CROWS) * (_WCOLS // _CCOLS)
_CHUNK = _CROWS * _CCOLS

_E5 = float(np.exp(5.0))


def _table_body(total_ref, counts_ref, out_ref):
    k = _E5 / total_ref[0, 0]
    out_ref[...] = 1.0 / (counts_ref[...] * k + (1.0 + 1e-9 * _E5))


def _transform_table(counts_padded, total_tokens):
    blk = _TROWS // _TGRID
    return pl.pallas_call(
        _table_body,
        grid=(_TGRID,),
        in_specs=[
            pl.BlockSpec(memory_space=pltpu.SMEM),
            pl.BlockSpec((blk, _LANES), lambda i: (i, 0)),
        ],
        out_specs=pl.BlockSpec((blk, _LANES), lambda i: (i, 0)),
        out_shape=jax.ShapeDtypeStruct((_TROWS, _LANES), jnp.float32),
    )(jnp.reshape(total_tokens, (1, 1)), counts_padded)


def _gather_body(xt_hbm, table_hbm, out_hbm, tab_s,
                 idx0, idx1, val0, val1,
                 tsem, isem0, isem1, gsem, osem0, osem1):
    sid = lax.axis_index("s")
    wid = sid * _NC + lax.axis_index("c")
    cbase = wid * _WCOLS

    idx_bufs, val_bufs = [idx0, idx1], [val0, val1]
    isems, osems = [isem0, isem1], [osem0, osem1]

    @pl.when(sid == 0)
    def _():
        pltpu.async_copy(table_hbm, tab_s, tsem).wait()

    def blk(c):
        return (pl.ds((c // 2) * _CROWS, _CROWS),
                pl.ds(cbase + (c % 2) * _CCOLS, _CCOLS))

    idx_loads = [
        pltpu.make_async_copy(xt_hbm.at[blk(c)], idx_bufs[c % 2], isems[c % 2])
        for c in range(_NCHUNKS)
    ]
    out_stores = [
        pltpu.make_async_copy(val_bufs[c % 2], out_hbm.at[blk(c)], osems[c % 2])
        for c in range(_NCHUNKS)
    ]

    idx_loads[0].start()
    plsc.subcore_barrier()

    for c in range(_NCHUNKS):
        b = c % 2
        if c >= 2:
            out_stores[c - 2].wait()
        idx_loads[c].wait()
        if c + 1 < _NCHUNKS:
            idx_loads[c + 1].start()

        def _fire(i, carry):
            r = i >> 1
            k = (i & 1) * 128
            pltpu.make_async_copy(
                tab_s.at[idx_bufs[b].at[r, pl.ds(k, 128)]],
                val_bufs[b].at[r, pl.ds(k, 128)], gsem).start()
            return carry

        lax.fori_loop(0, _CROWS * 2, _fire, 0, unroll=8)

        pltpu.make_async_copy(out_hbm.at[blk(c)], val_bufs[b], gsem).wait()
        out_stores[c].start()

    out_stores[_NCHUNKS - 2].wait()
    out_stores[_NCHUNKS - 1].wait()


_gather = pl.kernel(
    _gather_body,
    out_type=jax.ShapeDtypeStruct((_N, _B), jnp.float32),
    mesh=plsc.VectorSubcoreMesh(core_axis_name="c", subcore_axis_name="s"),
    scratch_types=[
        pltpu.VMEM_SHARED((_VOCAB_PAD,), jnp.float32),
        pltpu.VMEM((_CROWS, _CCOLS), jnp.int32),
        pltpu.VMEM((_CROWS, _CCOLS), jnp.int32),
        pltpu.VMEM((_CROWS, _CCOLS), jnp.float32),
        pltpu.VMEM((_CROWS, _CCOLS), jnp.float32),
        pltpu.SemaphoreType.DMA,
        pltpu.SemaphoreType.DMA,
        pltpu.SemaphoreType.DMA,
        pltpu.SemaphoreType.DMA,
        pltpu.SemaphoreType.DMA,
        pltpu.SemaphoreType.DMA,
    ],
)


@jax.jit
def kernel(x, token_counts, total_tokens):
    counts_padded = jnp.concatenate(
        [token_counts, jnp.ones((_VOCAB_PAD - _VOCAB,), jnp.float32)]
    ).reshape(_TROWS, _LANES)
    table = _transform_table(counts_padded, total_tokens).reshape(-1)
    out_t = _gather(x.T, table)
    return out_t.T


# speedup vs baseline: 1.0510x; 1.0510x over previous
import functools

import jax
import jax.numpy as jnp
import numpy as np
from jax import lax
from jax.experimental import pallas as pl
from jax.experimental.pallas import tpu as pltpu
from jax.experimental.pallas import tpu_sc as plsc

_VOCAB = 1_000_000
_VOCAB_PAD = 1_048_576
_TROWS = 8192
_TGRID = 8
_LANES = 128

_B, _N = 16384, 200
_TOTAL = _B * _N
_NC, _NS = 2, 16
_NW = _NC * _NS
_WCOLS = _B // _NW
_CROWS = 40
_CCOLS = 256
_NCHUNKS = (_N // _CROWS) * (_WCOLS // _CCOLS)
_CHUNK = _CROWS * _CCOLS

_E5 = float(np.exp(5.0))


def _table_body(total_ref, counts_ref, out_ref):
    total = total_ref[0, 0]
    freq = counts_ref[...] / total
    out_ref[...] = 1.0 / (1.0 + (freq + 1e-9) * _E5)


def _transform_table(counts_padded, total_tokens):
    blk = _TROWS // _TGRID
    return pl.pallas_call(
        _table_body,
        grid=(_TGRID,),
        in_specs=[
            pl.BlockSpec(memory_space=pltpu.SMEM),
            pl.BlockSpec((blk, _LANES), lambda i: (i, 0)),
        ],
        out_specs=pl.BlockSpec((blk, _LANES), lambda i: (i, 0)),
        out_shape=jax.ShapeDtypeStruct((_TROWS, _LANES), jnp.float32),
    )(jnp.reshape(total_tokens, (1, 1)), counts_padded)


def _gather_body(xt_hbm, table_hbm, out_hbm, tab_s,
                 idx0, idx1, val0, val1,
                 tsem, isem0, isem1, gsem, osem0, osem1):
    sid = lax.axis_index("s")
    wid = sid * _NC + lax.axis_index("c")
    cbase = wid * _WCOLS

    idx_bufs, val_bufs = [idx0, idx1], [val0, val1]
    isems, osems = [isem0, isem1], [osem0, osem1]

    @pl.when(sid == 0)
    def _():
        pltpu.async_copy(table_hbm, tab_s, tsem).wait()

    def blk(c):
        return (pl.ds((c // 2) * _CROWS, _CROWS),
                pl.ds(cbase + (c % 2) * _CCOLS, _CCOLS))

    idx_loads = [
        pltpu.make_async_copy(xt_hbm.at[blk(c)], idx_bufs[c % 2], isems[c % 2])
        for c in range(_NCHUNKS)
    ]
    out_stores = [
        pltpu.make_async_copy(val_bufs[c % 2], out_hbm.at[blk(c)], osems[c % 2])
        for c in range(_NCHUNKS)
    ]

    idx_loads[0].start()
    plsc.subcore_barrier()

    for c in range(_NCHUNKS):
        b = c % 2
        if c >= 2:
            out_stores[c - 2].wait()
        idx_loads[c].wait()
        if c + 1 < _NCHUNKS:
            idx_loads[c + 1].start()

        def _fire(i, carry):
            r = i >> 1
            k = (i & 1) * 128
            pltpu.make_async_copy(
                tab_s.at[idx_bufs[b].at[r, pl.ds(k, 128)]],
                val_bufs[b].at[r, pl.ds(k, 128)], gsem).start()
            return carry

        lax.fori_loop(0, _CROWS * 2, _fire, 0, unroll=8)

        def _drain(i, carry):
            pltpu.make_async_copy(
                tab_s.at[idx_bufs[b].at[0, pl.ds(0, 128)]],
                val_bufs[b].at[0, pl.ds(0, 128)], gsem).wait()
            return carry

        lax.fori_loop(0, _CROWS * 2, _drain, 0, unroll=8)
        out_stores[c].start()

    out_stores[_NCHUNKS - 2].wait()
    out_stores[_NCHUNKS - 1].wait()


_gather = pl.kernel(
    _gather_body,
    out_type=jax.ShapeDtypeStruct((_N, _B), jnp.float32),
    mesh=plsc.VectorSubcoreMesh(core_axis_name="c", subcore_axis_name="s"),
    scratch_types=[
        pltpu.VMEM_SHARED((_VOCAB_PAD,), jnp.float32),
        pltpu.VMEM((_CROWS, _CCOLS), jnp.int32),
        pltpu.VMEM((_CROWS, _CCOLS), jnp.int32),
        pltpu.VMEM((_CROWS, _CCOLS), jnp.float32),
        pltpu.VMEM((_CROWS, _CCOLS), jnp.float32),
        pltpu.SemaphoreType.DMA,
        pltpu.SemaphoreType.DMA,
        pltpu.SemaphoreType.DMA,
        pltpu.SemaphoreType.DMA,
        pltpu.SemaphoreType.DMA,
        pltpu.SemaphoreType.DMA,
    ],
)


@jax.jit
def kernel(x, token_counts, total_tokens):
    counts_padded = jnp.concatenate(
        [token_counts, jnp.ones((_VOCAB_PAD - _VOCAB,), jnp.float32)]
    ).reshape(_TROWS, _LANES)
    table = _transform_table(counts_padded, total_tokens).reshape(-1)
    out_t = _gather(x.T, table)
    return out_t.T
